# Initial kernel scaffold; baseline (speedup 1.0000x reference)
import jax
import jax.numpy as jnp
from jax.experimental import pallas as pl


def kernel(l3_xyz, l4_xyz, l3_points, l4_points, W1, b1, g1, be1, W2, b2, g2, be2):
    raise NotImplementedError("write your pallas kernel here")


# speedup vs baseline: 26.1089x; 26.1089x over previous
import functools

import jax
import jax.numpy as jnp
from jax.experimental import pallas as pl
from jax.experimental.pallas import tpu as pltpu

B, N, S = 8, 4096, 1024
CQ, CS = 256, 512
CIN, CMID, COUT = CQ + CS, 256, 256
M = B * N

TN_A = 512
TN_B = 2048


def _pass_a(x1_ref, x2_ref, l3_ref, l4p_ref, w1_ref, y1_ref, st_ref, p_ref):
    b = pl.program_id(0)
    nb = pl.program_id(1)

    @pl.when(nb == 0)
    def _():
        p_ref[...] = jax.lax.dot_general(
            w1_ref[:, CQ:], l4p_ref[0],
            (((1,), (0,)), ((), ())),
            preferred_element_type=jnp.float32)

    x1 = x1_ref[0]
    x2 = x2_ref[0]
    x1b = x1.astype(jnp.bfloat16).astype(jnp.float32)
    x2b = x2.astype(jnp.bfloat16).astype(jnp.float32)
    p0 = x1b[0][:, None] * x2b[0][None, :]
    p1 = x1b[1][:, None] * x2b[1][None, :]
    p2 = x1b[2][:, None] * x2b[2][None, :]
    s = p0 + p1
    bv = s - p0
    e1 = (p0 - (s - bv)) + (p1 - bv)
    t = s + p2
    cv = t - s
    e2 = (s - (t - cv)) + (p2 - cv)
    prod = t + (e1 + e2)
    x1sq = ((x1[0] * x1[0] + x1[1] * x1[1]) + x1[2] * x1[2])[:, None]
    x2sq = ((x2[0] * x2[0] + x2[1] * x2[1]) + x2[2] * x2[2])[None, :]
    d = (-2.0 * prod + x1sq) + x2sq

    iota = jax.lax.broadcasted_iota(jnp.int32, (TN_A, S), 1)
    big = jnp.float32(jnp.inf)

    m1 = jnp.min(d, axis=1, keepdims=True)
    i1 = jnp.min(jnp.where(d == m1, iota, S), axis=1, keepdims=True)
    dm = jnp.where(iota == i1, big, d)
    m2 = jnp.min(dm, axis=1, keepdims=True)
    i2 = jnp.min(jnp.where(dm == m2, iota, S), axis=1, keepdims=True)
    dm = jnp.where(iota == i2, big, dm)
    m3 = jnp.min(dm, axis=1, keepdims=True)
    i3 = jnp.min(jnp.where(dm == m3, iota, S), axis=1, keepdims=True)

    r1 = 1.0 / (m1 + 1e-8)
    r2 = 1.0 / (m2 + 1e-8)
    r3 = 1.0 / (m3 + 1e-8)
    nrm = r1 + r2 + r3
    wt = jnp.where(iota == i1, r1 / nrm,
                   jnp.where(iota == i2, r2 / nrm,
                             jnp.where(iota == i3, r3 / nrm, 0.0)))

    y = jax.lax.dot_general(
        w1_ref[:, :CQ], l3_ref[0],
        (((1,), (0,)), ((), ())),
        preferred_element_type=jnp.float32)
    y = y + jax.lax.dot_general(
        p_ref[...], wt,
        (((1,), (1,)), ((), ())),
        preferred_element_type=jnp.float32)
    y1_ref[0] = y

    @pl.when((b == 0) & (nb == 0))
    def _():
        st_ref[...] = jnp.zeros_like(st_ref)

    st_ref[0, :] += jnp.sum(y, axis=1)
    st_ref[1, :] += jnp.sum(y * y, axis=1)


def _pass_b(y1_ref, w2_ref, sc_ref, y2_ref, st_ref):
    b = pl.program_id(0)
    nb = pl.program_id(1)
    scale = sc_ref[0][:, None]
    shift = sc_ref[1][:, None]
    h = jnp.maximum(y1_ref[0] * scale + shift, 0.0)
    y = jax.lax.dot_general(
        w2_ref[...], h,
        (((1,), (0,)), ((), ())),
        preferred_element_type=jnp.float32)
    y2_ref[0] = y

    @pl.when((b == 0) & (nb == 0))
    def _():
        st_ref[...] = jnp.zeros_like(st_ref)

    st_ref[0, :] += jnp.sum(y, axis=1)
    st_ref[1, :] += jnp.sum(y * y, axis=1)


def _pass_c(y2_ref, sc_ref, out_ref):
    scale = sc_ref[0][:, None]
    shift = sc_ref[1][:, None]
    out_ref[0] = jnp.maximum(y2_ref[0] * scale + shift, 0.0)


def _scale_shift(st, g, be):
    mean = st[0] / M
    var = st[1] / M - mean * mean
    scale = g * jax.lax.rsqrt(var + 1e-5)
    shift = be - mean * scale
    return jnp.stack([scale, shift])


@jax.jit
def kernel(l3_xyz, l4_xyz, l3_points, l4_points, W1, b1, g1, be1, W2, b2, g2, be2):
    del b1, b2

    nb_a = N // TN_A
    y1, st1 = pl.pallas_call(
        _pass_a,
        grid=(B, nb_a),
        in_specs=[
            pl.BlockSpec((1, 3, TN_A), lambda b, n: (b, 0, n)),
            pl.BlockSpec((1, 3, S), lambda b, n: (b, 0, 0)),
            pl.BlockSpec((1, CQ, TN_A), lambda b, n: (b, 0, n)),
            pl.BlockSpec((1, CS, S), lambda b, n: (b, 0, 0)),
            pl.BlockSpec((CMID, CIN), lambda b, n: (0, 0)),
        ],
        out_specs=[
            pl.BlockSpec((1, CMID, TN_A), lambda b, n: (b, 0, n)),
            pl.BlockSpec((2, CMID), lambda b, n: (0, 0)),
        ],
        out_shape=[
            jax.ShapeDtypeStruct((B, CMID, N), jnp.float32),
            jax.ShapeDtypeStruct((2, CMID), jnp.float32),
        ],
        scratch_shapes=[pltpu.VMEM((CMID, S), jnp.float32)],
    )(l3_xyz, l4_xyz, l3_points, l4_points, W1)

    sc1 = _scale_shift(st1, g1, be1)

    nb_b = N // TN_B
    y2, st2 = pl.pallas_call(
        _pass_b,
        grid=(B, nb_b),
        in_specs=[
            pl.BlockSpec((1, CMID, TN_B), lambda b, n: (b, 0, n)),
            pl.BlockSpec((COUT, CMID), lambda b, n: (0, 0)),
            pl.BlockSpec((2, CMID), lambda b, n: (0, 0)),
        ],
        out_specs=[
            pl.BlockSpec((1, COUT, TN_B), lambda b, n: (b, 0, n)),
            pl.BlockSpec((2, COUT), lambda b, n: (0, 0)),
        ],
        out_shape=[
            jax.ShapeDtypeStruct((B, COUT, N), jnp.float32),
            jax.ShapeDtypeStruct((2, COUT), jnp.float32),
        ],
    )(y1, W2, sc1)

    sc2 = _scale_shift(st2, g2, be2)

    out = pl.pallas_call(
        _pass_c,
        grid=(B,),
        in_specs=[
            pl.BlockSpec((1, COUT, N), lambda b: (b, 0, 0)),
            pl.BlockSpec((2, COUT), lambda b: (0, 0)),
        ],
        out_specs=pl.BlockSpec((1, COUT, N), lambda b: (b, 0, 0)),
        out_shape=jax.ShapeDtypeStruct((B, COUT, N), jnp.float32),
    )(y2, sc2)

    return out


# speedup vs baseline: 41.9040x; 1.6050x over previous
import functools

import jax
import jax.numpy as jnp
from jax.experimental import pallas as pl
from jax.experimental.pallas import tpu as pltpu

B, N, S = 8, 4096, 1024
CQ, CS = 256, 512
CIN, CMID, COUT = CQ + CS, 256, 256
M = B * N

TN_A = 512
TN_B = 2048


def _pass_a(x1_ref, x2_ref, l3_ref, l4p_ref, w1_ref, y1_ref, st_ref, p_ref):
    b = pl.program_id(0)
    nb = pl.program_id(1)

    @pl.when(nb == 0)
    def _():
        p_ref[...] = jax.lax.dot_general(
            w1_ref[:, CQ:], l4p_ref[0],
            (((1,), (0,)), ((), ())),
            preferred_element_type=jnp.float32).astype(jnp.bfloat16)

    x1 = x1_ref[0]
    x2 = x2_ref[0]
    prod = jax.lax.dot_general(
        x1.astype(jnp.bfloat16), x2.astype(jnp.bfloat16),
        (((1,), (0,)), ((), ())),
        preferred_element_type=jnp.float32)
    x1sq = ((x1[:, 0] * x1[:, 0] + x1[:, 1] * x1[:, 1])
            + x1[:, 2] * x1[:, 2])[:, None]
    x2sq = ((x2[0] * x2[0] + x2[1] * x2[1]) + x2[2] * x2[2])[None, :]
    d = (-2.0 * prod + x1sq) + x2sq

    big = jnp.float32(jnp.inf)
    m1 = jnp.min(d, axis=1, keepdims=True)
    dm = jnp.where(d == m1, big, d)
    m2 = jnp.min(dm, axis=1, keepdims=True)
    dm = jnp.where(dm == m2, big, dm)
    m3 = jnp.min(dm, axis=1, keepdims=True)

    r1 = 1.0 / (m1 + 1e-8)
    r2 = 1.0 / (m2 + 1e-8)
    r3 = 1.0 / (m3 + 1e-8)
    nrm = r1 + r2 + r3
    wt = jnp.where(d == m1, r1 / nrm,
                   jnp.where(d == m2, r2 / nrm,
                             jnp.where(d == m3, r3 / nrm,
                                       0.0))).astype(jnp.bfloat16)

    y = jax.lax.dot_general(
        w1_ref[:, :CQ], l3_ref[0],
        (((1,), (0,)), ((), ())),
        preferred_element_type=jnp.float32)
    y = y + jax.lax.dot_general(
        p_ref[...], wt,
        (((1,), (1,)), ((), ())),
        preferred_element_type=jnp.float32)
    y1_ref[0] = y

    @pl.when((b == 0) & (nb == 0))
    def _():
        st_ref[...] = jnp.zeros_like(st_ref)

    st_ref[0, :] += jnp.sum(y, axis=1)
    st_ref[1, :] += jnp.sum(y * y, axis=1)


def _pass_b(y1_ref, w2_ref, sc_ref, y2_ref, st_ref):
    b = pl.program_id(0)
    nb = pl.program_id(1)
    scale = sc_ref[0][:, None]
    shift = sc_ref[1][:, None]
    h = jnp.maximum(y1_ref[0] * scale + shift, 0.0)
    y = jax.lax.dot_general(
        w2_ref[...], h,
        (((1,), (0,)), ((), ())),
        preferred_element_type=jnp.float32)
    y2_ref[0] = y

    @pl.when((b == 0) & (nb == 0))
    def _():
        st_ref[...] = jnp.zeros_like(st_ref)

    st_ref[0, :] += jnp.sum(y, axis=1)
    st_ref[1, :] += jnp.sum(y * y, axis=1)


def _pass_c(y2_ref, sc_ref, out_ref):
    scale = sc_ref[0][:, None]
    shift = sc_ref[1][:, None]
    out_ref[0] = jnp.maximum(y2_ref[0] * scale + shift, 0.0)


def _scale_shift(st, g, be):
    mean = st[0] / M
    var = st[1] / M - mean * mean
    scale = g * jax.lax.rsqrt(var + 1e-5)
    shift = be - mean * scale
    return jnp.stack([scale, shift])


@jax.jit
def kernel(l3_xyz, l4_xyz, l3_points, l4_points, W1, b1, g1, be1, W2, b2, g2, be2):
    del b1, b2

    l3_xyz_t = jnp.transpose(l3_xyz, (0, 2, 1))

    nb_a = N // TN_A
    y1, st1 = pl.pallas_call(
        _pass_a,
        grid=(B, nb_a),
        in_specs=[
            pl.BlockSpec((1, TN_A, 3), lambda b, n: (b, n, 0)),
            pl.BlockSpec((1, 3, S), lambda b, n: (b, 0, 0)),
            pl.BlockSpec((1, CQ, TN_A), lambda b, n: (b, 0, n)),
            pl.BlockSpec((1, CS, S), lambda b, n: (b, 0, 0)),
            pl.BlockSpec((CMID, CIN), lambda b, n: (0, 0)),
        ],
        out_specs=[
            pl.BlockSpec((1, CMID, TN_A), lambda b, n: (b, 0, n)),
            pl.BlockSpec((2, CMID), lambda b, n: (0, 0)),
        ],
        out_shape=[
            jax.ShapeDtypeStruct((B, CMID, N), jnp.float32),
            jax.ShapeDtypeStruct((2, CMID), jnp.float32),
        ],
        scratch_shapes=[pltpu.VMEM((CMID, S), jnp.bfloat16)],
    )(l3_xyz_t, l4_xyz, l3_points, l4_points, W1)

    sc1 = _scale_shift(st1, g1, be1)

    nb_b = N // TN_B
    y2, st2 = pl.pallas_call(
        _pass_b,
        grid=(B, nb_b),
        in_specs=[
            pl.BlockSpec((1, CMID, TN_B), lambda b, n: (b, 0, n)),
            pl.BlockSpec((COUT, CMID), lambda b, n: (0, 0)),
            pl.BlockSpec((2, CMID), lambda b, n: (0, 0)),
        ],
        out_specs=[
            pl.BlockSpec((1, COUT, TN_B), lambda b, n: (b, 0, n)),
            pl.BlockSpec((2, COUT), lambda b, n: (0, 0)),
        ],
        out_shape=[
            jax.ShapeDtypeStruct((B, COUT, N), jnp.float32),
            jax.ShapeDtypeStruct((2, COUT), jnp.float32),
        ],
    )(y1, W2, sc1)

    sc2 = _scale_shift(st2, g2, be2)

    out = pl.pallas_call(
        _pass_c,
        grid=(B,),
        in_specs=[
            pl.BlockSpec((1, COUT, N), lambda b: (b, 0, 0)),
            pl.BlockSpec((2, COUT), lambda b: (0, 0)),
        ],
        out_specs=pl.BlockSpec((1, COUT, N), lambda b: (b, 0, 0)),
        out_shape=jax.ShapeDtypeStruct((B, COUT, N), jnp.float32),
    )(y2, sc2)

    return out


# speedup vs baseline: 42.5834x; 1.0162x over previous
import functools

import jax
import jax.numpy as jnp
from jax.experimental import pallas as pl
from jax.experimental.pallas import tpu as pltpu

B, N, S = 8, 4096, 1024
CQ, CS = 256, 512
CIN, CMID, COUT = CQ + CS, 256, 256
M = B * N

TN_A = 512
TN_B = 2048


def _pass_a(x1_ref, x2_ref, l3_ref, l4p_ref, w1_ref, y1_ref, st_ref, p_ref):
    b = pl.program_id(0)
    nb = pl.program_id(1)

    @pl.when(nb == 0)
    def _():
        p_ref[...] = jax.lax.dot_general(
            w1_ref[:, CQ:], l4p_ref[0],
            (((1,), (0,)), ((), ())),
            preferred_element_type=jnp.float32).astype(jnp.bfloat16)

    x1 = x1_ref[0]
    x2 = x2_ref[0]
    prod = jax.lax.dot_general(
        x1.astype(jnp.bfloat16), x2.astype(jnp.bfloat16),
        (((1,), (0,)), ((), ())),
        preferred_element_type=jnp.float32)
    x1sq = ((x1[:, 0] * x1[:, 0] + x1[:, 1] * x1[:, 1])
            + x1[:, 2] * x1[:, 2])[:, None]
    x2sq = ((x2[0] * x2[0] + x2[1] * x2[1]) + x2[2] * x2[2])[None, :]
    d = (-2.0 * prod + x1sq) + x2sq

    big = jnp.float32(jnp.inf)
    m1 = jnp.min(d, axis=1, keepdims=True)
    dm = jnp.where(d == m1, big, d)
    m2 = jnp.min(dm, axis=1, keepdims=True)
    dm = jnp.where(dm == m2, big, dm)
    m3 = jnp.min(dm, axis=1, keepdims=True)

    r1 = 1.0 / (m1 + 1e-8)
    r2 = 1.0 / (m2 + 1e-8)
    r3 = 1.0 / (m3 + 1e-8)
    rn = 1.0 / (r1 + r2 + r3)
    wt = jnp.where(d <= m3, (1.0 / (d + 1e-8)) * rn,
                   0.0).astype(jnp.bfloat16)

    y = jax.lax.dot_general(
        w1_ref[:, :CQ], l3_ref[0],
        (((1,), (0,)), ((), ())),
        preferred_element_type=jnp.float32)
    y = y + jax.lax.dot_general(
        p_ref[...], wt,
        (((1,), (1,)), ((), ())),
        preferred_element_type=jnp.float32)
    y1_ref[0] = y

    @pl.when((b == 0) & (nb == 0))
    def _():
        st_ref[...] = jnp.zeros_like(st_ref)

    st_ref[0, :] += jnp.sum(y, axis=1)
    st_ref[1, :] += jnp.sum(y * y, axis=1)


def _pass_b(y1_ref, w2_ref, sc_ref, y2_ref, st_ref):
    b = pl.program_id(0)
    nb = pl.program_id(1)
    scale = sc_ref[0][:, None]
    shift = sc_ref[1][:, None]
    h = jnp.maximum(y1_ref[0] * scale + shift, 0.0)
    y = jax.lax.dot_general(
        w2_ref[...], h,
        (((1,), (0,)), ((), ())),
        preferred_element_type=jnp.float32)
    y2_ref[0] = y

    @pl.when((b == 0) & (nb == 0))
    def _():
        st_ref[...] = jnp.zeros_like(st_ref)

    st_ref[0, :] += jnp.sum(y, axis=1)
    st_ref[1, :] += jnp.sum(y * y, axis=1)


def _pass_c(y2_ref, sc_ref, out_ref):
    scale = sc_ref[0][:, None]
    shift = sc_ref[1][:, None]
    out_ref[0] = jnp.maximum(y2_ref[0] * scale + shift, 0.0)


def _scale_shift(st, g, be):
    mean = st[0] / M
    var = st[1] / M - mean * mean
    scale = g * jax.lax.rsqrt(var + 1e-5)
    shift = be - mean * scale
    return jnp.stack([scale, shift])


@jax.jit
def kernel(l3_xyz, l4_xyz, l3_points, l4_points, W1, b1, g1, be1, W2, b2, g2, be2):
    del b1, b2

    l3_xyz_t = jnp.transpose(l3_xyz, (0, 2, 1))

    nb_a = N // TN_A
    y1, st1 = pl.pallas_call(
        _pass_a,
        grid=(B, nb_a),
        in_specs=[
            pl.BlockSpec((1, TN_A, 3), lambda b, n: (b, n, 0)),
            pl.BlockSpec((1, 3, S), lambda b, n: (b, 0, 0)),
            pl.BlockSpec((1, CQ, TN_A), lambda b, n: (b, 0, n)),
            pl.BlockSpec((1, CS, S), lambda b, n: (b, 0, 0)),
            pl.BlockSpec((CMID, CIN), lambda b, n: (0, 0)),
        ],
        out_specs=[
            pl.BlockSpec((1, CMID, TN_A), lambda b, n: (b, 0, n)),
            pl.BlockSpec((2, CMID), lambda b, n: (0, 0)),
        ],
        out_shape=[
            jax.ShapeDtypeStruct((B, CMID, N), jnp.float32),
            jax.ShapeDtypeStruct((2, CMID), jnp.float32),
        ],
        scratch_shapes=[pltpu.VMEM((CMID, S), jnp.bfloat16)],
    )(l3_xyz_t, l4_xyz, l3_points, l4_points, W1)

    sc1 = _scale_shift(st1, g1, be1)

    nb_b = N // TN_B
    y2, st2 = pl.pallas_call(
        _pass_b,
        grid=(B, nb_b),
        in_specs=[
            pl.BlockSpec((1, CMID, TN_B), lambda b, n: (b, 0, n)),
            pl.BlockSpec((COUT, CMID), lambda b, n: (0, 0)),
            pl.BlockSpec((2, CMID), lambda b, n: (0, 0)),
        ],
        out_specs=[
            pl.BlockSpec((1, COUT, TN_B), lambda b, n: (b, 0, n)),
            pl.BlockSpec((2, COUT), lambda b, n: (0, 0)),
        ],
        out_shape=[
            jax.ShapeDtypeStruct((B, COUT, N), jnp.float32),
            jax.ShapeDtypeStruct((2, COUT), jnp.float32),
        ],
    )(y1, W2, sc1)

    sc2 = _scale_shift(st2, g2, be2)

    out = pl.pallas_call(
        _pass_c,
        grid=(B,),
        in_specs=[
            pl.BlockSpec((1, COUT, N), lambda b: (b, 0, 0)),
            pl.BlockSpec((2, COUT), lambda b: (0, 0)),
        ],
        out_specs=pl.BlockSpec((1, COUT, N), lambda b: (b, 0, 0)),
        out_shape=jax.ShapeDtypeStruct((B, COUT, N), jnp.float32),
    )(y2, sc2)

    return out


# speedup vs baseline: 45.2617x; 1.0629x over previous
import functools

import jax
import jax.numpy as jnp
from jax.experimental import pallas as pl
from jax.experimental.pallas import tpu as pltpu

B, N, S = 8, 4096, 1024
CQ, CS = 256, 512
CIN, CMID, COUT = CQ + CS, 256, 256
M = B * N

TN_A = 1024
TN_B = 2048


def _pass_a(x1_ref, x2_ref, l3_ref, l4p_ref, w1_ref, y1_ref, st_ref, p_ref):
    b = pl.program_id(0)
    nb = pl.program_id(1)

    @pl.when(nb == 0)
    def _():
        p_ref[...] = jax.lax.dot_general(
            w1_ref[:, CQ:], l4p_ref[0],
            (((1,), (0,)), ((), ())),
            preferred_element_type=jnp.float32).astype(jnp.bfloat16)

    x1 = x1_ref[0]
    x2 = x2_ref[0]
    prod = jax.lax.dot_general(
        x1.astype(jnp.bfloat16), x2.astype(jnp.bfloat16),
        (((1,), (0,)), ((), ())),
        preferred_element_type=jnp.float32)
    x1sq = ((x1[:, 0] * x1[:, 0] + x1[:, 1] * x1[:, 1])
            + x1[:, 2] * x1[:, 2])[:, None]
    x2sq = ((x2[0] * x2[0] + x2[1] * x2[1]) + x2[2] * x2[2])[None, :]
    d = (-2.0 * prod + x1sq) + x2sq

    big = jnp.float32(jnp.inf)
    m1 = jnp.min(d, axis=1, keepdims=True)
    dm = jnp.where(d == m1, big, d)
    m2 = jnp.min(dm, axis=1, keepdims=True)
    dm = jnp.where(dm == m2, big, dm)
    m3 = jnp.min(dm, axis=1, keepdims=True)

    r1 = 1.0 / (m1 + 1e-8)
    r2 = 1.0 / (m2 + 1e-8)
    r3 = 1.0 / (m3 + 1e-8)
    rn = 1.0 / (r1 + r2 + r3)
    wt = jnp.where(d <= m3, (1.0 / (d + 1e-8)) * rn,
                   0.0).astype(jnp.bfloat16)

    y = jax.lax.dot_general(
        w1_ref[:, :CQ], l3_ref[0],
        (((1,), (0,)), ((), ())),
        preferred_element_type=jnp.float32)
    y = y + jax.lax.dot_general(
        p_ref[...], wt,
        (((1,), (1,)), ((), ())),
        preferred_element_type=jnp.float32)
    y1_ref[0] = y

    @pl.when((b == 0) & (nb == 0))
    def _():
        st_ref[...] = jnp.zeros_like(st_ref)

    st_ref[0, :] += jnp.sum(y, axis=1)
    st_ref[1, :] += jnp.sum(y * y, axis=1)


def _pass_b(y1_ref, w2_ref, sc_ref, y2_ref, st_ref):
    b = pl.program_id(0)
    nb = pl.program_id(1)
    scale = sc_ref[0][:, None]
    shift = sc_ref[1][:, None]
    h = jnp.maximum(y1_ref[0] * scale + shift, 0.0)
    y = jax.lax.dot_general(
        w2_ref[...], h,
        (((1,), (0,)), ((), ())),
        preferred_element_type=jnp.float32)
    y2_ref[0] = y

    @pl.when((b == 0) & (nb == 0))
    def _():
        st_ref[...] = jnp.zeros_like(st_ref)

    st_ref[0, :] += jnp.sum(y, axis=1)
    st_ref[1, :] += jnp.sum(y * y, axis=1)


def _pass_c(y2_ref, sc_ref, out_ref):
    scale = sc_ref[0][:, None]
    shift = sc_ref[1][:, None]
    out_ref[0] = jnp.maximum(y2_ref[0] * scale + shift, 0.0)


def _scale_shift(st, g, be):
    mean = st[0] / M
    var = st[1] / M - mean * mean
    scale = g * jax.lax.rsqrt(var + 1e-5)
    shift = be - mean * scale
    return jnp.stack([scale, shift])


@jax.jit
def kernel(l3_xyz, l4_xyz, l3_points, l4_points, W1, b1, g1, be1, W2, b2, g2, be2):
    del b1, b2

    l3_xyz_t = jnp.transpose(l3_xyz, (0, 2, 1))

    nb_a = N // TN_A
    y1, st1 = pl.pallas_call(
        _pass_a,
        grid=(B, nb_a),
        in_specs=[
            pl.BlockSpec((1, TN_A, 3), lambda b, n: (b, n, 0)),
            pl.BlockSpec((1, 3, S), lambda b, n: (b, 0, 0)),
            pl.BlockSpec((1, CQ, TN_A), lambda b, n: (b, 0, n)),
            pl.BlockSpec((1, CS, S), lambda b, n: (b, 0, 0)),
            pl.BlockSpec((CMID, CIN), lambda b, n: (0, 0)),
        ],
        out_specs=[
            pl.BlockSpec((1, CMID, TN_A), lambda b, n: (b, 0, n)),
            pl.BlockSpec((2, CMID), lambda b, n: (0, 0)),
        ],
        out_shape=[
            jax.ShapeDtypeStruct((B, CMID, N), jnp.float32),
            jax.ShapeDtypeStruct((2, CMID), jnp.float32),
        ],
        scratch_shapes=[pltpu.VMEM((CMID, S), jnp.bfloat16)],
    )(l3_xyz_t, l4_xyz, l3_points, l4_points, W1)

    sc1 = _scale_shift(st1, g1, be1)

    nb_b = N // TN_B
    y2, st2 = pl.pallas_call(
        _pass_b,
        grid=(B, nb_b),
        in_specs=[
            pl.BlockSpec((1, CMID, TN_B), lambda b, n: (b, 0, n)),
            pl.BlockSpec((COUT, CMID), lambda b, n: (0, 0)),
            pl.BlockSpec((2, CMID), lambda b, n: (0, 0)),
        ],
        out_specs=[
            pl.BlockSpec((1, COUT, TN_B), lambda b, n: (b, 0, n)),
            pl.BlockSpec((2, COUT), lambda b, n: (0, 0)),
        ],
        out_shape=[
            jax.ShapeDtypeStruct((B, COUT, N), jnp.float32),
            jax.ShapeDtypeStruct((2, COUT), jnp.float32),
        ],
    )(y1, W2, sc1)

    sc2 = _scale_shift(st2, g2, be2)

    out = pl.pallas_call(
        _pass_c,
        grid=(B,),
        in_specs=[
            pl.BlockSpec((1, COUT, N), lambda b: (b, 0, 0)),
            pl.BlockSpec((2, COUT), lambda b: (0, 0)),
        ],
        out_specs=pl.BlockSpec((1, COUT, N), lambda b: (b, 0, 0)),
        out_shape=jax.ShapeDtypeStruct((B, COUT, N), jnp.float32),
    )(y2, sc2)

    return out
